# Initial kernel scaffold; baseline (speedup 1.0000x reference)
import jax
import jax.numpy as jnp
from jax.experimental import pallas as pl


def kernel(hidden_states, gate_w, w1, w2, w3):
    raise NotImplementedError("write your pallas kernel here")


# speedup vs baseline: 3.9049x; 3.9049x over previous
import jax
import jax.numpy as jnp
from jax.experimental import pallas as pl
from jax.experimental.pallas import tpu as pltpu

_TOKENS = 2048
_HIDDEN = 1024
_INTER = 4096
_TM = 1024
_BI = 512


def _ffn_body(x_ref, w1_ref, w3_ref, w2_ref, y_ref):
    j = pl.program_id(1)
    xb = x_ref[...].astype(jnp.bfloat16)
    w1b = w1_ref[...].astype(jnp.bfloat16)
    w3b = w3_ref[...].astype(jnp.bfloat16)
    w2b = w2_ref[...].astype(jnp.bfloat16)
    dims = (((1,), (1,)), ((), ()))
    h1 = jax.lax.dot_general(xb, w1b, dims, preferred_element_type=jnp.float32)
    h3 = jax.lax.dot_general(xb, w3b, dims, preferred_element_type=jnp.float32)
    g = (jax.nn.silu(h1) * h3).astype(jnp.bfloat16)
    contrib = jax.lax.dot_general(g, w2b, dims, preferred_element_type=jnp.float32)

    @pl.when(j == 0)
    def _init():
        y_ref[...] = contrib

    @pl.when(j > 0)
    def _acc():
        y_ref[...] += contrib


def kernel(hidden_states, gate_w, w1, w2, w3):
    del gate_w
    w1e = w1[0]
    w3e = w3[0]
    w2e = w2[0]
    grid = (_TOKENS // _TM, _INTER // _BI)
    return pl.pallas_call(
        _ffn_body,
        grid=grid,
        in_specs=[
            pl.BlockSpec((_TM, _HIDDEN), lambda t, j: (t, 0)),
            pl.BlockSpec((_BI, _HIDDEN), lambda t, j: (j, 0)),
            pl.BlockSpec((_BI, _HIDDEN), lambda t, j: (j, 0)),
            pl.BlockSpec((_HIDDEN, _BI), lambda t, j: (0, j)),
        ],
        out_specs=pl.BlockSpec((_TM, _HIDDEN), lambda t, j: (t, 0)),
        out_shape=jax.ShapeDtypeStruct((_TOKENS, _HIDDEN), jnp.float32),
        compiler_params=pltpu.CompilerParams(
            dimension_semantics=("arbitrary", "arbitrary"),
        ),
    )(hidden_states, w1e, w3e, w2e)


# speedup vs baseline: 4.0558x; 1.0386x over previous
import jax
import jax.numpy as jnp
from jax.experimental import pallas as pl
from jax.experimental.pallas import tpu as pltpu

_TOKENS = 2048
_HIDDEN = 1024
_INTER = 4096
_TM = 1024
_BI = 512
_NI = _INTER // _BI
_DIMS = (((1,), (1,)), ((), ()))


def _ffn_body(x_ref, w1_ref, w3_ref, w2_ref, y_ref, xb_s, g_s, w2b_s):
    t = pl.program_id(0)
    j = pl.program_id(1)

    @pl.when(j == 0)
    def _cast_x():
        xb_s[...] = x_ref[...].astype(jnp.bfloat16)

    @pl.when(t == 0)
    def _cast_w2():
        w2b_s[:, pl.ds(j * _BI, _BI)] = w2_ref[...].astype(jnp.bfloat16)

    xb = xb_s[...]
    w1b = w1_ref[...].astype(jnp.bfloat16)
    w3b = w3_ref[...].astype(jnp.bfloat16)
    h1 = jax.lax.dot_general(xb, w1b, _DIMS, preferred_element_type=jnp.float32)
    h3 = jax.lax.dot_general(xb, w3b, _DIMS, preferred_element_type=jnp.float32)
    g_s[:, pl.ds(j * _BI, _BI)] = (jax.nn.silu(h1) * h3).astype(jnp.bfloat16)

    @pl.when(j == _NI - 1)
    def _project():
        y_ref[...] = jax.lax.dot_general(
            g_s[...], w2b_s[...], _DIMS, preferred_element_type=jnp.float32
        )


def kernel(hidden_states, gate_w, w1, w2, w3):
    del gate_w
    grid = (_TOKENS // _TM, _NI)
    return pl.pallas_call(
        _ffn_body,
        grid=grid,
        in_specs=[
            pl.BlockSpec((_TM, _HIDDEN), lambda t, j: (t, 0)),
            pl.BlockSpec((_BI, _HIDDEN), lambda t, j: (j, 0)),
            pl.BlockSpec((_BI, _HIDDEN), lambda t, j: (j, 0)),
            pl.BlockSpec((_HIDDEN, _BI), lambda t, j: (0, j)),
        ],
        out_specs=pl.BlockSpec((_TM, _HIDDEN), lambda t, j: (t, 0)),
        out_shape=jax.ShapeDtypeStruct((_TOKENS, _HIDDEN), jnp.float32),
        scratch_shapes=[
            pltpu.VMEM((_TM, _HIDDEN), jnp.bfloat16),
            pltpu.VMEM((_TM, _INTER), jnp.bfloat16),
            pltpu.VMEM((_HIDDEN, _INTER), jnp.bfloat16),
        ],
        compiler_params=pltpu.CompilerParams(
            dimension_semantics=("arbitrary", "arbitrary"),
        ),
    )(hidden_states, w1[0], w3[0], w2[0])
